# baseline (speedup 1.0000x reference)
import jax
import jax.numpy as jnp
from jax import lax
from jax.experimental import pallas as pl
from jax.experimental.pallas import tpu as pltpu
from jax.experimental.pallas import tpu_sc as plsc

N = 10000
E = 320000
D = 128
H = 128
C = 40

NC = 2
NS = 16
NW = NC * NS
FW = H // NC
NPAD = 16000
STRIPE = NPAD // NS
ZR = 50
DW = 16

EPT = E // NS
KS = 50
NCHS = EPT // KS
NBUF = 4

EPW = E // NW
KD = 125
NCHD = EPW // KD


def _zero_rows(ref, rows, cols):
    def body(i, carry):
        for j in range(cols // 16):
            ref[i, pl.ds(j * 16, 16)] = jnp.zeros((16,), jnp.float32)
        return carry
    lax.fori_loop(0, rows, body, 0)


def _deg_body(dst_hbm, out_hbm, dstv, onesv, zbuf, acc_sh):
    c = lax.axis_index("c")
    s = lax.axis_index("s")
    wid = c * NS + s

    def fill_ones(i, carry):
        onesv[i, pl.ds(0, 16)] = jnp.ones((16,), jnp.float32)
        return carry
    lax.fori_loop(0, KD, fill_ones, 0)
    _zero_rows(zbuf, ZR, DW)
    base = s * STRIPE
    for t in range(STRIPE // ZR):
        pltpu.sync_copy(zbuf, acc_sh.at[pl.ds(base + t * ZR, ZR)])
    pltpu.sync_copy(dst_hbm.at[wid], dstv)
    plsc.subcore_barrier()

    def chunk(j, carry):
        pltpu.sync_copy(onesv, acc_sh.at[dstv.at[j]], add=True)
        return carry
    lax.fori_loop(0, NCHD, chunk, 0)
    plsc.subcore_barrier()
    pltpu.sync_copy(acc_sh.at[pl.ds(base, STRIPE)],
                    out_hbm.at[pl.ds(base, STRIPE), pl.ds(c * FW, DW)])


def _seg_body(zs_hbm, src_hbm, dst_hbm, out_hbm,
              srcv, dstv, rows, gsems, ssem, zbuf, acc_sh):
    c = lax.axis_index("c")
    s = lax.axis_index("s")

    _zero_rows(zbuf, ZR, FW)
    base = s * STRIPE
    for t in range(STRIPE // ZR):
        pltpu.sync_copy(zbuf, acc_sh.at[pl.ds(base + t * ZR, ZR)])
    pltpu.sync_copy(src_hbm.at[c * NS + s], srcv)
    pltpu.sync_copy(dst_hbm.at[s], dstv)
    plsc.subcore_barrier()

    def wait_gather(j, b):
        pltpu.make_async_copy(zs_hbm.at[srcv.at[j]], rows.at[b],
                              gsems.at[b]).wait()

    def fire_gather(j, b):
        pltpu.async_copy(zs_hbm.at[srcv.at[j]], rows.at[b], gsems.at[b])

    def fire_scatter(j, b):
        pltpu.async_copy(rows.at[b], acc_sh.at[dstv.at[j]], ssem, add=True)

    def wait_scatter(j, b):
        pltpu.make_async_copy(rows.at[b], acc_sh.at[dstv.at[j]], ssem).wait()

    fire_gather(0, 0)
    fire_gather(1, 1)
    for u in range(2):
        wait_gather(u, u)
        fire_scatter(u, u)
        fire_gather(u + 2, u + 2)

    def block(blk, carry):
        j0 = 2 + blk * NBUF
        for u in range(NBUF):
            j = j0 + u
            b = (2 + u) % NBUF
            wait_gather(j, b)
            fire_scatter(j, b)
            bp = u % NBUF
            wait_scatter(j - 2, bp)
            fire_gather(j + 2, bp)
        return carry
    lax.fori_loop(0, (NCHS - 4) // NBUF, block, 0)

    for u in range(2):
        j = NCHS - 2 + u
        b = j % NBUF
        wait_gather(j, b)
        fire_scatter(j, b)
        wait_scatter(j - 2, (j - 2) % NBUF)
    for u in range(2):
        j = NCHS - 2 + u
        wait_scatter(j, j % NBUF)

    plsc.subcore_barrier()
    pltpu.sync_copy(acc_sh.at[pl.ds(base, STRIPE)],
                    out_hbm.at[pl.ds(base, STRIPE), pl.ds(c * FW, FW)])


def _sc_mesh():
    return plsc.VectorSubcoreMesh(core_axis_name="c", subcore_axis_name="s",
                                  num_cores=NC, num_subcores=NS)


def _deg_call(dst3):
    fn = pl.kernel(
        _deg_body,
        out_type=jax.ShapeDtypeStruct((NPAD, H), jnp.float32),
        mesh=_sc_mesh(),
        scratch_types=[
            pltpu.VMEM((NCHD, KD), jnp.int32),
            pltpu.VMEM((KD, DW), jnp.float32),
            pltpu.VMEM((ZR, DW), jnp.float32),
            pltpu.VMEM_SHARED((NPAD, DW), jnp.float32),
        ],
        compiler_params=pltpu.CompilerParams(use_tc_tiling_on_sc=False),
        name="sc_degree",
    )
    return fn(dst3)


def _seg_call(zs2n, src4, dst3):
    fn = pl.kernel(
        _seg_body,
        out_type=jax.ShapeDtypeStruct((NPAD, H), jnp.float32),
        mesh=_sc_mesh(),
        scratch_types=[
            pltpu.VMEM((NCHS, KS), jnp.int32),
            pltpu.VMEM((NCHS, KS), jnp.int32),
            pltpu.VMEM((NBUF, KS, FW), jnp.float32),
            pltpu.SemaphoreType.DMA((NBUF,)),
            pltpu.SemaphoreType.DMA,
            pltpu.VMEM((ZR, FW), jnp.float32),
            pltpu.VMEM_SHARED((NPAD, FW), jnp.float32),
        ],
        compiler_params=pltpu.CompilerParams(use_tc_tiling_on_sc=False),
        name="sc_segsum",
    )
    return fn(zs2n, src4, dst3)


B = 2000
GRID = N // B
_MM = (((1,), (1,)), ((), ()))


def _row_spec(cols):
    return pl.BlockSpec((B, cols), lambda i: (i, 0))


def _full_spec(r, cols):
    return pl.BlockSpec((r, cols), lambda i: (0, 0))


def _a_body(degp, x, w, dis_o, zs_o):
    d = degp[...]
    deg = d[:, 0:1] + d[:, FW:FW + 1] + 1.0
    dis = lax.rsqrt(deg)
    z = lax.dot_general(x[...], w[...], _MM, preferred_element_type=jnp.float32)
    dis_o[...] = jnp.broadcast_to(dis, (B, H))
    zs_o[...] = dis * z


def _b_body(u, zs, dis, b, w, h_o, zs1_o):
    h = jnp.maximum(dis[...] * (u[...] + zs[...]) + b[...], 0.0)
    h_o[...] = h
    z1 = lax.dot_general(h, w[...], _MM, preferred_element_type=jnp.float32)
    zs1_o[...] = dis[...] * z1


def _c_body(u, zs, dis, b, h0, w, zs2_o):
    h1 = jnp.maximum(dis[...] * (u[...] + zs[...]) + b[...], 0.0) + h0[...]
    z2 = lax.dot_general(h1, w[...], _MM, preferred_element_type=jnp.float32)
    zs2_o[...] = dis[...] * z2


def _d_body(u, zs, dis, b, wm, bm, out_o):
    h2 = dis[...] * (u[...] + zs[...]) + b[...]
    out_o[...] = (
        lax.dot_general(h2, wm[...], _MM, preferred_element_type=jnp.float32)
        + bm[...]
    )


def _stage_a(degp, x, w0):
    return pl.pallas_call(
        _a_body,
        grid=(GRID,),
        in_specs=[_row_spec(H), _row_spec(D), _full_spec(H, D)],
        out_specs=[_row_spec(H), _row_spec(H)],
        out_shape=[jax.ShapeDtypeStruct((N, H), jnp.float32),
                   jax.ShapeDtypeStruct((N, H), jnp.float32)],
    )(degp, x, w0)


def _stage_b(u, zs, dis, b0, w1):
    return pl.pallas_call(
        _b_body,
        grid=(GRID,),
        in_specs=[_row_spec(H), _row_spec(H), _row_spec(H),
                  _full_spec(1, H), _full_spec(H, H)],
        out_specs=[_row_spec(H), _row_spec(H)],
        out_shape=[jax.ShapeDtypeStruct((N, H), jnp.float32),
                   jax.ShapeDtypeStruct((N, H), jnp.float32)],
    )(u, zs, dis, b0, w1)


def _stage_c(u, zs, dis, b1, h0, w2):
    return pl.pallas_call(
        _c_body,
        grid=(GRID,),
        in_specs=[_row_spec(H), _row_spec(H), _row_spec(H),
                  _full_spec(1, H), _row_spec(H), _full_spec(H, H)],
        out_specs=_row_spec(H),
        out_shape=jax.ShapeDtypeStruct((N, H), jnp.float32),
    )(u, zs, dis, b1, h0, w2)


def _stage_d(u, zs, dis, b2, wm, bm):
    return pl.pallas_call(
        _d_body,
        grid=(GRID,),
        in_specs=[_row_spec(H), _row_spec(H), _row_spec(H),
                  _full_spec(1, H), _full_spec(C, H), _full_spec(1, C)],
        out_specs=pl.BlockSpec((B, C), lambda i: (i, 0)),
        out_shape=jax.ShapeDtypeStruct((N, C), jnp.float32),
    )(u, zs, dis, b2, wm, bm)


def kernel(X, A, W0, b0, W1, b1, W2, b2, Wm, bm):
    src, dst = A[0], A[1]
    src_t = (2 * src).reshape(NS, NCHS, KS)
    src4 = jnp.concatenate([src_t, src_t + 1], axis=0)
    dst3s = dst.reshape(NS, NCHS, KS)
    dst3d = dst.reshape(NW, NCHD, KD)

    degp = _deg_call(dst3d)
    dis_b, zs0 = _stage_a(degp, X, W0)

    u0 = _seg_call(zs0.reshape(2 * N, FW), src4, dst3s)
    h0, zs1 = _stage_b(u0, zs0, dis_b, b0.reshape(1, H), W1)

    u1 = _seg_call(zs1.reshape(2 * N, FW), src4, dst3s)
    zs2 = _stage_c(u1, zs1, dis_b, b1.reshape(1, H), h0, W2)

    u2 = _seg_call(zs2.reshape(2 * N, FW), src4, dst3s)
    return _stage_d(u2, zs2, dis_b, b2.reshape(1, H), Wm, bm.reshape(1, C))


# speedup vs baseline: 1.4095x; 1.4095x over previous
import jax
import jax.numpy as jnp
from jax import lax
from jax.experimental import pallas as pl
from jax.experimental.pallas import tpu as pltpu
from jax.experimental.pallas import tpu_sc as plsc

N = 10000
E = 320000
D = 128
H = 128
C = 40

NC = 2
NS = 16
NW = NC * NS
FW = H // NC
NPAD = 10112
STRIPE = NPAD // NS
ZR = 79
DW = 16

EPT = E // NS
KS = 125
NCHS = EPT // KS
NBUF = 4

EPW = E // NW
KD = 125
NCHD = EPW // KD


def _zero_rows(ref, rows, cols):
    def body(i, carry):
        for j in range(cols // 16):
            ref[i, pl.ds(j * 16, 16)] = jnp.zeros((16,), jnp.float32)
        return carry
    lax.fori_loop(0, rows, body, 0)


def _deg_body(dst_hbm, out_hbm, dstv, onesv, zbuf, acc_sh):
    c = lax.axis_index("c")
    s = lax.axis_index("s")
    wid = c * NS + s

    def fill_ones(i, carry):
        onesv[i, pl.ds(0, 16)] = jnp.ones((16,), jnp.float32)
        return carry
    lax.fori_loop(0, KD, fill_ones, 0)
    _zero_rows(zbuf, ZR, DW)
    base = s * STRIPE
    for t in range(STRIPE // ZR):
        pltpu.sync_copy(zbuf, acc_sh.at[pl.ds(base + t * ZR, ZR)])
    pltpu.sync_copy(dst_hbm.at[wid], dstv)
    plsc.subcore_barrier()

    def chunk(j, carry):
        pltpu.sync_copy(onesv, acc_sh.at[dstv.at[j]], add=True)
        return carry
    lax.fori_loop(0, NCHD, chunk, 0)
    plsc.subcore_barrier()
    pltpu.sync_copy(acc_sh.at[pl.ds(base, STRIPE)],
                    out_hbm.at[pl.ds(base, STRIPE), pl.ds(c * FW, DW)])


def _seg_body(zs_hbm, src_hbm, dst_hbm, out_hbm,
              srcv, dstv, rows, gsems, ssem, zbuf, acc_sh):
    c = lax.axis_index("c")
    s = lax.axis_index("s")

    _zero_rows(zbuf, ZR, FW)
    base = s * STRIPE
    for t in range(STRIPE // ZR):
        pltpu.sync_copy(zbuf, acc_sh.at[pl.ds(base + t * ZR, ZR)])
    pltpu.sync_copy(src_hbm.at[c * NS + s], srcv)
    pltpu.sync_copy(dst_hbm.at[s], dstv)
    plsc.subcore_barrier()

    def wait_gather(j, b):
        pltpu.make_async_copy(zs_hbm.at[srcv.at[j]], rows.at[b],
                              gsems.at[b]).wait()

    def fire_gather(j, b):
        pltpu.async_copy(zs_hbm.at[srcv.at[j]], rows.at[b], gsems.at[b])

    def fire_scatter(j, b):
        pltpu.async_copy(rows.at[b], acc_sh.at[dstv.at[j]], ssem, add=True)

    def wait_scatter(j, b):
        pltpu.make_async_copy(rows.at[b], acc_sh.at[dstv.at[j]], ssem).wait()

    fire_gather(0, 0)
    fire_gather(1, 1)
    for u in range(2):
        wait_gather(u, u)
        fire_scatter(u, u)
        fire_gather(u + 2, u + 2)

    def block(blk, carry):
        j0 = 2 + blk * NBUF
        for u in range(NBUF):
            j = j0 + u
            b = (2 + u) % NBUF
            wait_gather(j, b)
            fire_scatter(j, b)
            bp = u % NBUF
            wait_scatter(j - 2, bp)
            fire_gather(j + 2, bp)
        return carry
    lax.fori_loop(0, (NCHS - 4) // NBUF, block, 0)

    for u in range(2):
        j = NCHS - 2 + u
        b = j % NBUF
        wait_gather(j, b)
        fire_scatter(j, b)
        wait_scatter(j - 2, (j - 2) % NBUF)
    for u in range(2):
        j = NCHS - 2 + u
        wait_scatter(j, j % NBUF)

    plsc.subcore_barrier()
    pltpu.sync_copy(acc_sh.at[pl.ds(base, STRIPE)],
                    out_hbm.at[pl.ds(base, STRIPE), pl.ds(c * FW, FW)])


def _sc_mesh():
    return plsc.VectorSubcoreMesh(core_axis_name="c", subcore_axis_name="s",
                                  num_cores=NC, num_subcores=NS)


def _deg_call(dst3):
    fn = pl.kernel(
        _deg_body,
        out_type=jax.ShapeDtypeStruct((NPAD, H), jnp.float32),
        mesh=_sc_mesh(),
        scratch_types=[
            pltpu.VMEM((NCHD, KD), jnp.int32),
            pltpu.VMEM((KD, DW), jnp.float32),
            pltpu.VMEM((ZR, DW), jnp.float32),
            pltpu.VMEM_SHARED((NPAD, DW), jnp.float32),
        ],
        compiler_params=pltpu.CompilerParams(use_tc_tiling_on_sc=False),
        name="sc_degree",
    )
    return fn(dst3)


def _seg_call(zs2n, src4, dst3):
    fn = pl.kernel(
        _seg_body,
        out_type=jax.ShapeDtypeStruct((NPAD, H), jnp.float32),
        mesh=_sc_mesh(),
        scratch_types=[
            pltpu.VMEM((NCHS, KS), jnp.int32),
            pltpu.VMEM((NCHS, KS), jnp.int32),
            pltpu.VMEM((NBUF, KS, FW), jnp.float32),
            pltpu.SemaphoreType.DMA((NBUF,)),
            pltpu.SemaphoreType.DMA,
            pltpu.VMEM((ZR, FW), jnp.float32),
            pltpu.VMEM_SHARED((NPAD, FW), jnp.float32),
        ],
        compiler_params=pltpu.CompilerParams(use_tc_tiling_on_sc=False),
        name="sc_segsum",
    )
    return fn(zs2n, src4, dst3)


B = 2000
GRID = N // B
_MM = (((1,), (1,)), ((), ()))


def _row_spec(cols):
    return pl.BlockSpec((B, cols), lambda i: (i, 0))


def _full_spec(r, cols):
    return pl.BlockSpec((r, cols), lambda i: (0, 0))


def _a_body(degp, x, w, dis_o, zs_o):
    d = degp[...]
    deg = d[:, 0:1] + d[:, FW:FW + 1] + 1.0
    dis = lax.rsqrt(deg)
    z = lax.dot_general(x[...], w[...], _MM, preferred_element_type=jnp.float32)
    dis_o[...] = jnp.broadcast_to(dis, (B, H))
    zs_o[...] = dis * z


def _b_body(u, zs, dis, b, w, h_o, zs1_o):
    h = jnp.maximum(dis[...] * (u[...] + zs[...]) + b[...], 0.0)
    h_o[...] = h
    z1 = lax.dot_general(h, w[...], _MM, preferred_element_type=jnp.float32)
    zs1_o[...] = dis[...] * z1


def _c_body(u, zs, dis, b, h0, w, zs2_o):
    h1 = jnp.maximum(dis[...] * (u[...] + zs[...]) + b[...], 0.0) + h0[...]
    z2 = lax.dot_general(h1, w[...], _MM, preferred_element_type=jnp.float32)
    zs2_o[...] = dis[...] * z2


def _d_body(u, zs, dis, b, wm, bm, out_o):
    h2 = dis[...] * (u[...] + zs[...]) + b[...]
    out_o[...] = (
        lax.dot_general(h2, wm[...], _MM, preferred_element_type=jnp.float32)
        + bm[...]
    )


def _stage_a(degp, x, w0):
    return pl.pallas_call(
        _a_body,
        grid=(GRID,),
        in_specs=[_row_spec(H), _row_spec(D), _full_spec(H, D)],
        out_specs=[_row_spec(H), _row_spec(H)],
        out_shape=[jax.ShapeDtypeStruct((N, H), jnp.float32),
                   jax.ShapeDtypeStruct((N, H), jnp.float32)],
    )(degp, x, w0)


def _stage_b(u, zs, dis, b0, w1):
    return pl.pallas_call(
        _b_body,
        grid=(GRID,),
        in_specs=[_row_spec(H), _row_spec(H), _row_spec(H),
                  _full_spec(1, H), _full_spec(H, H)],
        out_specs=[_row_spec(H), _row_spec(H)],
        out_shape=[jax.ShapeDtypeStruct((N, H), jnp.float32),
                   jax.ShapeDtypeStruct((N, H), jnp.float32)],
    )(u, zs, dis, b0, w1)


def _stage_c(u, zs, dis, b1, h0, w2):
    return pl.pallas_call(
        _c_body,
        grid=(GRID,),
        in_specs=[_row_spec(H), _row_spec(H), _row_spec(H),
                  _full_spec(1, H), _row_spec(H), _full_spec(H, H)],
        out_specs=_row_spec(H),
        out_shape=jax.ShapeDtypeStruct((N, H), jnp.float32),
    )(u, zs, dis, b1, h0, w2)


def _stage_d(u, zs, dis, b2, wm, bm):
    return pl.pallas_call(
        _d_body,
        grid=(GRID,),
        in_specs=[_row_spec(H), _row_spec(H), _row_spec(H),
                  _full_spec(1, H), _full_spec(C, H), _full_spec(1, C)],
        out_specs=pl.BlockSpec((B, C), lambda i: (i, 0)),
        out_shape=jax.ShapeDtypeStruct((N, C), jnp.float32),
    )(u, zs, dis, b2, wm, bm)


def kernel(X, A, W0, b0, W1, b1, W2, b2, Wm, bm):
    src, dst = A[0], A[1]
    src_t = (2 * src).reshape(NS, NCHS, KS)
    src4 = jnp.concatenate([src_t, src_t + 1], axis=0)
    dst3s = dst.reshape(NS, NCHS, KS)
    dst3d = dst.reshape(NW, NCHD, KD)

    degp = _deg_call(dst3d)
    dis_b, zs0 = _stage_a(degp, X, W0)

    u0 = _seg_call(zs0.reshape(2 * N, FW), src4, dst3s)
    h0, zs1 = _stage_b(u0, zs0, dis_b, b0.reshape(1, H), W1)

    u1 = _seg_call(zs1.reshape(2 * N, FW), src4, dst3s)
    zs2 = _stage_c(u1, zs1, dis_b, b1.reshape(1, H), h0, W2)

    u2 = _seg_call(zs2.reshape(2 * N, FW), src4, dst3s)
    return _stage_d(u2, zs2, dis_b, b2.reshape(1, H), Wm, bm.reshape(1, C))


# speedup vs baseline: 1.4212x; 1.0083x over previous
import jax
import jax.numpy as jnp
from jax import lax
from jax.experimental import pallas as pl
from jax.experimental.pallas import tpu as pltpu
from jax.experimental.pallas import tpu_sc as plsc

N = 10000
E = 320000
D = 128
H = 128
C = 40

NC = 2
NS = 16
NW = NC * NS
FW = H // NC
NPAD = 10112
STRIPE = NPAD // NS
ZR = 79
DW = 16

EPT = E // NS
KS = 125
NCHS = EPT // KS
NBUF = 4

EPW = E // NW
KD = 125
NCHD = EPW // KD


def _zero_rows(ref, rows, cols):
    def body(i, carry):
        for j in range(cols // 16):
            ref[i, pl.ds(j * 16, 16)] = jnp.zeros((16,), jnp.float32)
        return carry
    lax.fori_loop(0, rows, body, 0)


def _deg_body(dst_hbm, out_hbm, dstv, onesv, zbuf, acc_sh):
    c = lax.axis_index("c")
    s = lax.axis_index("s")
    wid = c * NS + s

    def fill_ones(i, carry):
        onesv[i, pl.ds(0, 16)] = jnp.ones((16,), jnp.float32)
        return carry
    lax.fori_loop(0, KD, fill_ones, 0)
    _zero_rows(zbuf, ZR, DW)
    base = s * STRIPE
    for t in range(STRIPE // ZR):
        pltpu.sync_copy(zbuf, acc_sh.at[pl.ds(base + t * ZR, ZR)])
    pltpu.sync_copy(dst_hbm.at[wid], dstv)
    plsc.subcore_barrier()

    def chunk(j, carry):
        pltpu.sync_copy(onesv, acc_sh.at[dstv.at[j]], add=True)
        return carry
    lax.fori_loop(0, NCHD, chunk, 0)
    plsc.subcore_barrier()
    pltpu.sync_copy(acc_sh.at[pl.ds(base, STRIPE)],
                    out_hbm.at[pl.ds(base, STRIPE), pl.ds(c * FW, DW)])


def _seg_body(zs_hbm, src_hbm, dst_hbm, out_hbm,
              srcv, dstv, rows, gsems, ssem, zbuf, acc_sh):
    c = lax.axis_index("c")
    s = lax.axis_index("s")

    pltpu.sync_copy(src_hbm.at[c * NS + s], srcv)
    pltpu.sync_copy(dst_hbm.at[s], dstv)

    def wait_gather(j, b):
        pltpu.make_async_copy(zs_hbm.at[srcv.at[j]], rows.at[b],
                              gsems.at[b]).wait()

    def fire_gather(j, b):
        pltpu.async_copy(zs_hbm.at[srcv.at[j]], rows.at[b], gsems.at[b])

    def fire_scatter(j, b):
        pltpu.async_copy(rows.at[b], acc_sh.at[dstv.at[j]], ssem, add=True)

    def wait_scatter(j, b):
        pltpu.make_async_copy(rows.at[b], acc_sh.at[dstv.at[j]], ssem).wait()

    fire_gather(0, 0)
    fire_gather(1, 1)
    _zero_rows(zbuf, ZR, FW)
    base = s * STRIPE
    for t in range(STRIPE // ZR):
        pltpu.sync_copy(zbuf, acc_sh.at[pl.ds(base + t * ZR, ZR)])
    plsc.subcore_barrier()
    for u in range(2):
        wait_gather(u, u)
        fire_scatter(u, u)
        fire_gather(u + 2, u + 2)

    def block(blk, carry):
        j0 = 2 + blk * NBUF
        for u in range(NBUF):
            j = j0 + u
            b = (2 + u) % NBUF
            wait_gather(j, b)
            fire_scatter(j, b)
            bp = u % NBUF
            wait_scatter(j - 2, bp)
            fire_gather(j + 2, bp)
        return carry
    lax.fori_loop(0, (NCHS - 4) // NBUF, block, 0)

    for u in range(2):
        j = NCHS - 2 + u
        b = j % NBUF
        wait_gather(j, b)
        fire_scatter(j, b)
        wait_scatter(j - 2, (j - 2) % NBUF)
    for u in range(2):
        j = NCHS - 2 + u
        wait_scatter(j, j % NBUF)

    plsc.subcore_barrier()
    pltpu.sync_copy(acc_sh.at[pl.ds(base, STRIPE)],
                    out_hbm.at[pl.ds(base, STRIPE), pl.ds(c * FW, FW)])


def _sc_mesh():
    return plsc.VectorSubcoreMesh(core_axis_name="c", subcore_axis_name="s",
                                  num_cores=NC, num_subcores=NS)


def _deg_call(dst3):
    fn = pl.kernel(
        _deg_body,
        out_type=jax.ShapeDtypeStruct((NPAD, H), jnp.float32),
        mesh=_sc_mesh(),
        scratch_types=[
            pltpu.VMEM((NCHD, KD), jnp.int32),
            pltpu.VMEM((KD, DW), jnp.float32),
            pltpu.VMEM((ZR, DW), jnp.float32),
            pltpu.VMEM_SHARED((NPAD, DW), jnp.float32),
        ],
        compiler_params=pltpu.CompilerParams(use_tc_tiling_on_sc=False),
        name="sc_degree",
    )
    return fn(dst3)


def _seg_call(zs2n, src4, dst3):
    fn = pl.kernel(
        _seg_body,
        out_type=jax.ShapeDtypeStruct((NPAD, H), jnp.float32),
        mesh=_sc_mesh(),
        scratch_types=[
            pltpu.VMEM((NCHS, KS), jnp.int32),
            pltpu.VMEM((NCHS, KS), jnp.int32),
            pltpu.VMEM((NBUF, KS, FW), jnp.float32),
            pltpu.SemaphoreType.DMA((NBUF,)),
            pltpu.SemaphoreType.DMA,
            pltpu.VMEM((ZR, FW), jnp.float32),
            pltpu.VMEM_SHARED((NPAD, FW), jnp.float32),
        ],
        compiler_params=pltpu.CompilerParams(use_tc_tiling_on_sc=False),
        name="sc_segsum",
    )
    return fn(zs2n, src4, dst3)


B = 2000
GRID = N // B
_MM = (((1,), (1,)), ((), ()))


def _row_spec(cols):
    return pl.BlockSpec((B, cols), lambda i: (i, 0))


def _full_spec(r, cols):
    return pl.BlockSpec((r, cols), lambda i: (0, 0))


def _a_body(degp, x, w, dis_o, zs_o):
    d = degp[...]
    deg = d[:, 0:1] + d[:, FW:FW + 1] + 1.0
    dis = lax.rsqrt(deg)
    z = lax.dot_general(x[...], w[...], _MM, preferred_element_type=jnp.float32)
    dis_o[...] = jnp.broadcast_to(dis, (B, H))
    zs_o[...] = dis * z


def _b_body(u, zs, dis, b, w, h_o, zs1_o):
    h = jnp.maximum(dis[...] * (u[...] + zs[...]) + b[...], 0.0)
    h_o[...] = h
    z1 = lax.dot_general(h, w[...], _MM, preferred_element_type=jnp.float32)
    zs1_o[...] = dis[...] * z1


def _c_body(u, zs, dis, b, h0, w, zs2_o):
    h1 = jnp.maximum(dis[...] * (u[...] + zs[...]) + b[...], 0.0) + h0[...]
    z2 = lax.dot_general(h1, w[...], _MM, preferred_element_type=jnp.float32)
    zs2_o[...] = dis[...] * z2


def _d_body(u, zs, dis, b, wm, bm, out_o):
    h2 = dis[...] * (u[...] + zs[...]) + b[...]
    out_o[...] = (
        lax.dot_general(h2, wm[...], _MM, preferred_element_type=jnp.float32)
        + bm[...]
    )


def _stage_a(degp, x, w0):
    return pl.pallas_call(
        _a_body,
        grid=(GRID,),
        in_specs=[_row_spec(H), _row_spec(D), _full_spec(H, D)],
        out_specs=[_row_spec(H), _row_spec(H)],
        out_shape=[jax.ShapeDtypeStruct((N, H), jnp.float32),
                   jax.ShapeDtypeStruct((N, H), jnp.float32)],
    )(degp, x, w0)


def _stage_b(u, zs, dis, b0, w1):
    return pl.pallas_call(
        _b_body,
        grid=(GRID,),
        in_specs=[_row_spec(H), _row_spec(H), _row_spec(H),
                  _full_spec(1, H), _full_spec(H, H)],
        out_specs=[_row_spec(H), _row_spec(H)],
        out_shape=[jax.ShapeDtypeStruct((N, H), jnp.float32),
                   jax.ShapeDtypeStruct((N, H), jnp.float32)],
    )(u, zs, dis, b0, w1)


def _stage_c(u, zs, dis, b1, h0, w2):
    return pl.pallas_call(
        _c_body,
        grid=(GRID,),
        in_specs=[_row_spec(H), _row_spec(H), _row_spec(H),
                  _full_spec(1, H), _row_spec(H), _full_spec(H, H)],
        out_specs=_row_spec(H),
        out_shape=jax.ShapeDtypeStruct((N, H), jnp.float32),
    )(u, zs, dis, b1, h0, w2)


def _stage_d(u, zs, dis, b2, wm, bm):
    return pl.pallas_call(
        _d_body,
        grid=(GRID,),
        in_specs=[_row_spec(H), _row_spec(H), _row_spec(H),
                  _full_spec(1, H), _full_spec(C, H), _full_spec(1, C)],
        out_specs=pl.BlockSpec((B, C), lambda i: (i, 0)),
        out_shape=jax.ShapeDtypeStruct((N, C), jnp.float32),
    )(u, zs, dis, b2, wm, bm)


def kernel(X, A, W0, b0, W1, b1, W2, b2, Wm, bm):
    src, dst = A[0], A[1]
    src_t = (2 * src).reshape(NS, NCHS, KS)
    src4 = jnp.concatenate([src_t, src_t + 1], axis=0)
    dst3s = dst.reshape(NS, NCHS, KS)
    dst3d = dst.reshape(NW, NCHD, KD)

    degp = _deg_call(dst3d)
    dis_b, zs0 = _stage_a(degp, X, W0)

    u0 = _seg_call(zs0.reshape(2 * N, FW), src4, dst3s)
    h0, zs1 = _stage_b(u0, zs0, dis_b, b0.reshape(1, H), W1)

    u1 = _seg_call(zs1.reshape(2 * N, FW), src4, dst3s)
    zs2 = _stage_c(u1, zs1, dis_b, b1.reshape(1, H), h0, W2)

    u2 = _seg_call(zs2.reshape(2 * N, FW), src4, dst3s)
    return _stage_d(u2, zs2, dis_b, b2.reshape(1, H), Wm, bm.reshape(1, C))


# speedup vs baseline: 1.4349x; 1.0097x over previous
import jax
import jax.numpy as jnp
from jax import lax
from jax.experimental import pallas as pl
from jax.experimental.pallas import tpu as pltpu
from jax.experimental.pallas import tpu_sc as plsc

N = 10000
E = 320000
D = 128
H = 128
C = 40

NC = 2
NS = 16
NW = NC * NS
FW = H // NC
NPAD = 10112
STRIPE = NPAD // NS
ZR = 79
DW = 16

EPT = E // NS
KS = 125
NCHS = EPT // KS
NBUF = 4

EPW = E // NW
KD = 125
NCHD = EPW // KD


def _zero_rows(ref, rows, cols):
    def body(i, carry):
        for j in range(cols // 16):
            ref[i, pl.ds(j * 16, 16)] = jnp.zeros((16,), jnp.float32)
        return carry
    lax.fori_loop(0, rows, body, 0)


def _deg_body(dst_hbm, out_hbm, dstv, onesv, zbuf, acc_sh):
    c = lax.axis_index("c")
    s = lax.axis_index("s")
    wid = c * NS + s

    def fill_ones(i, carry):
        onesv[i, pl.ds(0, 16)] = jnp.ones((16,), jnp.float32)
        return carry
    lax.fori_loop(0, KD, fill_ones, 0)
    _zero_rows(zbuf, ZR, DW)
    base = s * STRIPE
    for t in range(STRIPE // ZR):
        pltpu.sync_copy(zbuf, acc_sh.at[pl.ds(base + t * ZR, ZR)])
    pltpu.sync_copy(dst_hbm.at[wid], dstv)
    plsc.subcore_barrier()

    def chunk(j, carry):
        pltpu.sync_copy(onesv, acc_sh.at[dstv.at[j]], add=True)
        return carry
    lax.fori_loop(0, NCHD, chunk, 0)
    plsc.subcore_barrier()
    pltpu.sync_copy(acc_sh.at[pl.ds(base, STRIPE)],
                    out_hbm.at[pl.ds(base, STRIPE), pl.ds(c * FW, DW)])


def _seg_body(zs_hbm, src_hbm, dst_hbm, out_hbm,
              srcv, dstv, rows, gsems, ssem, zbuf, acc_sh):
    c = lax.axis_index("c")
    s = lax.axis_index("s")

    pltpu.sync_copy(src_hbm.at[c * NS + s], srcv)
    pltpu.sync_copy(dst_hbm.at[s], dstv)

    def wait_gather(j, b):
        pltpu.make_async_copy(zs_hbm.at[srcv.at[j]], rows.at[b],
                              gsems.at[b]).wait()

    def fire_gather(j, b):
        pltpu.async_copy(zs_hbm.at[srcv.at[j]], rows.at[b], gsems.at[b])

    def fire_scatter(j, b):
        pltpu.async_copy(rows.at[b], acc_sh.at[dstv.at[j]], ssem, add=True)

    def wait_scatter(j, b):
        pltpu.make_async_copy(rows.at[b], acc_sh.at[dstv.at[j]], ssem).wait()

    fire_gather(0, 0)
    fire_gather(1, 1)
    _zero_rows(zbuf, ZR, FW)
    base = s * STRIPE
    for t in range(STRIPE // ZR):
        pltpu.sync_copy(zbuf, acc_sh.at[pl.ds(base + t * ZR, ZR)])
    plsc.subcore_barrier()
    for u in range(2):
        wait_gather(u, u)
        fire_scatter(u, u)
        fire_gather(u + 2, u + 2)

    def block(blk, carry):
        j0 = 2 + blk * NBUF
        for u in range(NBUF):
            j = j0 + u
            b = (2 + u) % NBUF
            wait_gather(j, b)
            fire_scatter(j, b)
            bp = u % NBUF
            wait_scatter(j - 2, bp)
            fire_gather(j + 2, bp)
        return carry
    lax.fori_loop(0, (NCHS - 4) // NBUF, block, 0)

    for u in range(2):
        j = NCHS - 2 + u
        b = j % NBUF
        wait_gather(j, b)
        fire_scatter(j, b)
        wait_scatter(j - 2, (j - 2) % NBUF)
    for u in range(2):
        j = NCHS - 2 + u
        wait_scatter(j, j % NBUF)

    plsc.subcore_barrier()
    pltpu.sync_copy(acc_sh.at[pl.ds(base, STRIPE)],
                    out_hbm.at[pl.ds(base, STRIPE), pl.ds(c * FW, FW)])


def _sc_mesh():
    return plsc.VectorSubcoreMesh(core_axis_name="c", subcore_axis_name="s",
                                  num_cores=NC, num_subcores=NS)


def _deg_call(dst3):
    fn = pl.kernel(
        _deg_body,
        out_type=jax.ShapeDtypeStruct((NPAD, H), jnp.float32),
        mesh=_sc_mesh(),
        scratch_types=[
            pltpu.VMEM((NCHD, KD), jnp.int32),
            pltpu.VMEM((KD, DW), jnp.float32),
            pltpu.VMEM((ZR, DW), jnp.float32),
            pltpu.VMEM_SHARED((NPAD, DW), jnp.float32),
        ],
        compiler_params=pltpu.CompilerParams(use_tc_tiling_on_sc=False),
        name="sc_degree",
    )
    return fn(dst3)


def _seg_call(zs2n, src4, dst3):
    fn = pl.kernel(
        _seg_body,
        out_type=jax.ShapeDtypeStruct((NPAD, H), jnp.float32),
        mesh=_sc_mesh(),
        scratch_types=[
            pltpu.VMEM((NCHS, KS), jnp.int32),
            pltpu.VMEM((NCHS, KS), jnp.int32),
            pltpu.VMEM((NBUF, KS, FW), jnp.float32),
            pltpu.SemaphoreType.DMA((NBUF,)),
            pltpu.SemaphoreType.DMA,
            pltpu.VMEM((ZR, FW), jnp.float32),
            pltpu.VMEM_SHARED((NPAD, FW), jnp.float32),
        ],
        compiler_params=pltpu.CompilerParams(use_tc_tiling_on_sc=False),
        name="sc_segsum",
    )
    return fn(zs2n, src4, dst3)


B = 5000
GRID = N // B
_MM = (((1,), (1,)), ((), ()))


def _row_spec(cols):
    return pl.BlockSpec((B, cols), lambda i: (i, 0))


def _full_spec(r, cols):
    return pl.BlockSpec((r, cols), lambda i: (0, 0))


def _a_body(degp, x, w, dis_o, zs_o):
    d = degp[...]
    deg = d[:, 0:1] + d[:, FW:FW + 1] + 1.0
    dis = lax.rsqrt(deg)
    z = lax.dot_general(x[...], w[...], _MM, preferred_element_type=jnp.float32)
    dis_o[...] = jnp.broadcast_to(dis, (B, H))
    zs_o[...] = dis * z


def _b_body(u, zs, dis, b, w, h_o, zs1_o):
    h = jnp.maximum(dis[...] * (u[...] + zs[...]) + b[...], 0.0)
    h_o[...] = h
    z1 = lax.dot_general(h, w[...], _MM, preferred_element_type=jnp.float32)
    zs1_o[...] = dis[...] * z1


def _c_body(u, zs, dis, b, h0, w, zs2_o):
    h1 = jnp.maximum(dis[...] * (u[...] + zs[...]) + b[...], 0.0) + h0[...]
    z2 = lax.dot_general(h1, w[...], _MM, preferred_element_type=jnp.float32)
    zs2_o[...] = dis[...] * z2


def _d_body(u, zs, dis, b, wm, bm, out_o):
    h2 = dis[...] * (u[...] + zs[...]) + b[...]
    out_o[...] = (
        lax.dot_general(h2, wm[...], _MM, preferred_element_type=jnp.float32)
        + bm[...]
    )


def _stage_a(degp, x, w0):
    return pl.pallas_call(
        _a_body,
        grid=(GRID,),
        in_specs=[_row_spec(H), _row_spec(D), _full_spec(H, D)],
        out_specs=[_row_spec(H), _row_spec(H)],
        out_shape=[jax.ShapeDtypeStruct((N, H), jnp.float32),
                   jax.ShapeDtypeStruct((N, H), jnp.float32)],
    )(degp, x, w0)


def _stage_b(u, zs, dis, b0, w1):
    return pl.pallas_call(
        _b_body,
        grid=(GRID,),
        in_specs=[_row_spec(H), _row_spec(H), _row_spec(H),
                  _full_spec(1, H), _full_spec(H, H)],
        out_specs=[_row_spec(H), _row_spec(H)],
        out_shape=[jax.ShapeDtypeStruct((N, H), jnp.float32),
                   jax.ShapeDtypeStruct((N, H), jnp.float32)],
    )(u, zs, dis, b0, w1)


def _stage_c(u, zs, dis, b1, h0, w2):
    return pl.pallas_call(
        _c_body,
        grid=(GRID,),
        in_specs=[_row_spec(H), _row_spec(H), _row_spec(H),
                  _full_spec(1, H), _row_spec(H), _full_spec(H, H)],
        out_specs=_row_spec(H),
        out_shape=jax.ShapeDtypeStruct((N, H), jnp.float32),
    )(u, zs, dis, b1, h0, w2)


def _stage_d(u, zs, dis, b2, wm, bm):
    return pl.pallas_call(
        _d_body,
        grid=(GRID,),
        in_specs=[_row_spec(H), _row_spec(H), _row_spec(H),
                  _full_spec(1, H), _full_spec(C, H), _full_spec(1, C)],
        out_specs=pl.BlockSpec((B, C), lambda i: (i, 0)),
        out_shape=jax.ShapeDtypeStruct((N, C), jnp.float32),
    )(u, zs, dis, b2, wm, bm)


def kernel(X, A, W0, b0, W1, b1, W2, b2, Wm, bm):
    src, dst = A[0], A[1]
    src_t = (2 * src).reshape(NS, NCHS, KS)
    src4 = jnp.concatenate([src_t, src_t + 1], axis=0)
    dst3s = dst.reshape(NS, NCHS, KS)
    dst3d = dst.reshape(NW, NCHD, KD)

    degp = _deg_call(dst3d)
    dis_b, zs0 = _stage_a(degp, X, W0)

    u0 = _seg_call(zs0.reshape(2 * N, FW), src4, dst3s)
    h0, zs1 = _stage_b(u0, zs0, dis_b, b0.reshape(1, H), W1)

    u1 = _seg_call(zs1.reshape(2 * N, FW), src4, dst3s)
    zs2 = _stage_c(u1, zs1, dis_b, b1.reshape(1, H), h0, W2)

    u2 = _seg_call(zs2.reshape(2 * N, FW), src4, dst3s)
    return _stage_d(u2, zs2, dis_b, b2.reshape(1, H), Wm, bm.reshape(1, C))


# speedup vs baseline: 1.7326x; 1.2075x over previous
import jax
import jax.numpy as jnp
from jax import lax
from jax.experimental import pallas as pl
from jax.experimental.pallas import tpu as pltpu
from jax.experimental.pallas import tpu_sc as plsc

N = 10000
E = 320000
D = 128
H = 128
C = 40

NC = 2
NS = 16
NW = NC * NS
FW = H // NC
NPAD = 10112
STRIPE = NPAD // NS
ZR = 79
DW = 16

EPT = E // NS
KS = 125
NCHS = EPT // KS
NBUF = 4

EPW = E // NW
KD = 125
NCHD = EPW // KD


def _zero_rows(ref, rows, cols):
    def body(i, carry):
        for j in range(cols // 16):
            ref[i, pl.ds(j * 16, 16)] = jnp.zeros((16,), jnp.float32)
        return carry
    lax.fori_loop(0, rows, body, 0)


def _deg_body(dst_hbm, out_hbm, dstv, onesv, zbuf, acc_sh):
    c = lax.axis_index("c")
    s = lax.axis_index("s")
    wid = c * NS + s

    def fill_ones(i, carry):
        onesv[i, pl.ds(0, 16)] = jnp.ones((16,), jnp.float32)
        return carry
    lax.fori_loop(0, KD, fill_ones, 0)
    _zero_rows(zbuf, ZR, DW)
    base = s * STRIPE
    for t in range(STRIPE // ZR):
        pltpu.sync_copy(zbuf, acc_sh.at[pl.ds(base + t * ZR, ZR)])
    pltpu.sync_copy(dst_hbm.at[wid], dstv)
    plsc.subcore_barrier()

    def chunk(j, carry):
        pltpu.sync_copy(onesv, acc_sh.at[dstv.at[j]], add=True)
        return carry
    lax.fori_loop(0, NCHD, chunk, 0)
    plsc.subcore_barrier()
    pltpu.sync_copy(acc_sh.at[pl.ds(base, STRIPE)],
                    out_hbm.at[pl.ds(base, STRIPE), pl.ds(c * FW, DW)])


def _seg_body(zs_hbm, src_hbm, dst_hbm, out_hbm,
              srcv, dstv, rows, gsems, ssem, zbuf, acc_sh):
    c = lax.axis_index("c")
    s = lax.axis_index("s")

    pltpu.sync_copy(src_hbm.at[c * NS + s], srcv)
    pltpu.sync_copy(dst_hbm.at[s], dstv)

    def wait_gather(j, b):
        pltpu.make_async_copy(zs_hbm.at[srcv.at[j]], rows.at[b],
                              gsems.at[b]).wait()

    def fire_gather(j, b):
        pltpu.async_copy(zs_hbm.at[srcv.at[j]], rows.at[b], gsems.at[b])

    def fire_scatter(j, b):
        pltpu.async_copy(rows.at[b], acc_sh.at[dstv.at[j]], ssem, add=True)

    def wait_scatter(j, b):
        pltpu.make_async_copy(rows.at[b], acc_sh.at[dstv.at[j]], ssem).wait()

    fire_gather(0, 0)
    fire_gather(1, 1)
    _zero_rows(zbuf, ZR, FW)
    base = s * STRIPE
    for t in range(STRIPE // ZR):
        pltpu.sync_copy(zbuf, acc_sh.at[pl.ds(base + t * ZR, ZR)])
    plsc.subcore_barrier()
    for u in range(2):
        wait_gather(u, u)
        fire_scatter(u, u)
        fire_gather(u + 2, u + 2)

    def block(blk, carry):
        j0 = 2 + blk * NBUF
        for u in range(NBUF):
            j = j0 + u
            b = (2 + u) % NBUF
            bp = u % NBUF
            wait_scatter(j - 2, bp)
            fire_gather(j + 2, bp)
            wait_gather(j, b)
            fire_scatter(j, b)
        return carry
    lax.fori_loop(0, (NCHS - 4) // NBUF, block, 0)

    for u in range(2):
        j = NCHS - 2 + u
        b = j % NBUF
        wait_gather(j, b)
        fire_scatter(j, b)
        wait_scatter(j - 2, (j - 2) % NBUF)
    for u in range(2):
        j = NCHS - 2 + u
        wait_scatter(j, j % NBUF)

    plsc.subcore_barrier()
    pltpu.sync_copy(acc_sh.at[pl.ds(base, STRIPE)],
                    out_hbm.at[pl.ds(base, STRIPE), pl.ds(c * FW, FW)])


def _sc_mesh():
    return plsc.VectorSubcoreMesh(core_axis_name="c", subcore_axis_name="s",
                                  num_cores=NC, num_subcores=NS)


def _deg_call(dst3):
    fn = pl.kernel(
        _deg_body,
        out_type=jax.ShapeDtypeStruct((NPAD, H), jnp.float32),
        mesh=_sc_mesh(),
        scratch_types=[
            pltpu.VMEM((NCHD, KD), jnp.int32),
            pltpu.VMEM((KD, DW), jnp.float32),
            pltpu.VMEM((ZR, DW), jnp.float32),
            pltpu.VMEM_SHARED((NPAD, DW), jnp.float32),
        ],
        compiler_params=pltpu.CompilerParams(use_tc_tiling_on_sc=False),
        name="sc_degree",
    )
    return fn(dst3)


def _seg_call(zs2n, src4, dst3):
    fn = pl.kernel(
        _seg_body,
        out_type=jax.ShapeDtypeStruct((NPAD, H), jnp.float32),
        mesh=_sc_mesh(),
        scratch_types=[
            pltpu.VMEM((NCHS, KS), jnp.int32),
            pltpu.VMEM((NCHS, KS), jnp.int32),
            pltpu.VMEM((NBUF, KS, FW), jnp.float32),
            pltpu.SemaphoreType.DMA((NBUF,)),
            pltpu.SemaphoreType.DMA,
            pltpu.VMEM((ZR, FW), jnp.float32),
            pltpu.VMEM_SHARED((NPAD, FW), jnp.float32),
        ],
        compiler_params=pltpu.CompilerParams(use_tc_tiling_on_sc=False),
        name="sc_segsum",
    )
    return fn(zs2n, src4, dst3)


B = 5000
GRID = N // B
_MM = (((1,), (1,)), ((), ()))


def _row_spec(cols):
    return pl.BlockSpec((B, cols), lambda i: (i, 0))


def _full_spec(r, cols):
    return pl.BlockSpec((r, cols), lambda i: (0, 0))


def _a_body(degp, x, w, dis_o, zs_o):
    d = degp[...]
    deg = d[:, 0:1] + d[:, FW:FW + 1] + 1.0
    dis = lax.rsqrt(deg)
    z = lax.dot_general(x[...], w[...], _MM, preferred_element_type=jnp.float32)
    dis_o[...] = jnp.broadcast_to(dis, (B, H))
    zs_o[...] = dis * z


def _b_body(u, zs, dis, b, w, h_o, zs1_o):
    h = jnp.maximum(dis[...] * (u[...] + zs[...]) + b[...], 0.0)
    h_o[...] = h
    z1 = lax.dot_general(h, w[...], _MM, preferred_element_type=jnp.float32)
    zs1_o[...] = dis[...] * z1


def _c_body(u, zs, dis, b, h0, w, zs2_o):
    h1 = jnp.maximum(dis[...] * (u[...] + zs[...]) + b[...], 0.0) + h0[...]
    z2 = lax.dot_general(h1, w[...], _MM, preferred_element_type=jnp.float32)
    zs2_o[...] = dis[...] * z2


def _d_body(u, zs, dis, b, wm, bm, out_o):
    h2 = dis[...] * (u[...] + zs[...]) + b[...]
    out_o[...] = (
        lax.dot_general(h2, wm[...], _MM, preferred_element_type=jnp.float32)
        + bm[...]
    )


def _stage_a(degp, x, w0):
    return pl.pallas_call(
        _a_body,
        grid=(GRID,),
        in_specs=[_row_spec(H), _row_spec(D), _full_spec(H, D)],
        out_specs=[_row_spec(H), _row_spec(H)],
        out_shape=[jax.ShapeDtypeStruct((N, H), jnp.float32),
                   jax.ShapeDtypeStruct((N, H), jnp.float32)],
    )(degp, x, w0)


def _stage_b(u, zs, dis, b0, w1):
    return pl.pallas_call(
        _b_body,
        grid=(GRID,),
        in_specs=[_row_spec(H), _row_spec(H), _row_spec(H),
                  _full_spec(1, H), _full_spec(H, H)],
        out_specs=[_row_spec(H), _row_spec(H)],
        out_shape=[jax.ShapeDtypeStruct((N, H), jnp.float32),
                   jax.ShapeDtypeStruct((N, H), jnp.float32)],
    )(u, zs, dis, b0, w1)


def _stage_c(u, zs, dis, b1, h0, w2):
    return pl.pallas_call(
        _c_body,
        grid=(GRID,),
        in_specs=[_row_spec(H), _row_spec(H), _row_spec(H),
                  _full_spec(1, H), _row_spec(H), _full_spec(H, H)],
        out_specs=_row_spec(H),
        out_shape=jax.ShapeDtypeStruct((N, H), jnp.float32),
    )(u, zs, dis, b1, h0, w2)


def _stage_d(u, zs, dis, b2, wm, bm):
    return pl.pallas_call(
        _d_body,
        grid=(GRID,),
        in_specs=[_row_spec(H), _row_spec(H), _row_spec(H),
                  _full_spec(1, H), _full_spec(C, H), _full_spec(1, C)],
        out_specs=pl.BlockSpec((B, C), lambda i: (i, 0)),
        out_shape=jax.ShapeDtypeStruct((N, C), jnp.float32),
    )(u, zs, dis, b2, wm, bm)


def kernel(X, A, W0, b0, W1, b1, W2, b2, Wm, bm):
    src, dst = A[0], A[1]
    src_t = (2 * src).reshape(NS, NCHS, KS)
    src4 = jnp.concatenate([src_t, src_t + 1], axis=0)
    dst3s = dst.reshape(NS, NCHS, KS)
    dst3d = dst.reshape(NW, NCHD, KD)

    degp = _deg_call(dst3d)
    dis_b, zs0 = _stage_a(degp, X, W0)

    u0 = _seg_call(zs0.reshape(2 * N, FW), src4, dst3s)
    h0, zs1 = _stage_b(u0, zs0, dis_b, b0.reshape(1, H), W1)

    u1 = _seg_call(zs1.reshape(2 * N, FW), src4, dst3s)
    zs2 = _stage_c(u1, zs1, dis_b, b1.reshape(1, H), h0, W2)

    u2 = _seg_call(zs2.reshape(2 * N, FW), src4, dst3s)
    return _stage_d(u2, zs2, dis_b, b2.reshape(1, H), Wm, bm.reshape(1, C))


# speedup vs baseline: 1.7512x; 1.0107x over previous
import jax
import jax.numpy as jnp
from jax import lax
from jax.experimental import pallas as pl
from jax.experimental.pallas import tpu as pltpu
from jax.experimental.pallas import tpu_sc as plsc

N = 10000
E = 320000
D = 128
H = 128
C = 40

NC = 2
NS = 16
NW = NC * NS
FW = H // NC
NPAD = 10112
STRIPE = NPAD // NS
ZR = 79
DW = 16

EPT = E // NS
KS = 125
NCHS = EPT // KS
NBUF = 4

EPW = E // NW
KD = 125
NCHD = EPW // KD


def _zero_rows(ref, rows, cols):
    def body(i, carry):
        for j in range(cols // 16):
            ref[i, pl.ds(j * 16, 16)] = jnp.zeros((16,), jnp.float32)
        return carry
    lax.fori_loop(0, rows, body, 0)


def _deg_body(dst_hbm, out_hbm, dstv, onesv, zbuf, acc_sh, ssem):
    c = lax.axis_index("c")
    s = lax.axis_index("s")
    wid = c * NS + s

    pltpu.sync_copy(dst_hbm.at[wid], dstv)

    def fill_ones(i, carry):
        onesv[i, pl.ds(0, 16)] = jnp.ones((16,), jnp.float32)
        return carry
    lax.fori_loop(0, KD, fill_ones, 0)
    _zero_rows(zbuf, ZR, DW)
    base = s * STRIPE
    for t in range(STRIPE // ZR):
        pltpu.sync_copy(zbuf, acc_sh.at[pl.ds(base + t * ZR, ZR)])
    plsc.subcore_barrier()

    def chunk(j, carry):
        pltpu.async_copy(onesv, acc_sh.at[dstv.at[j]], ssem, add=True)
        return carry
    lax.fori_loop(0, NCHD, chunk, 0)

    def drain(j, carry):
        pltpu.make_async_copy(onesv, acc_sh.at[dstv.at[j]], ssem).wait()
        return carry
    lax.fori_loop(0, NCHD, drain, 0)
    plsc.subcore_barrier()
    pltpu.sync_copy(acc_sh.at[pl.ds(base, STRIPE)],
                    out_hbm.at[pl.ds(base, STRIPE), pl.ds(c * FW, DW)])


def _seg_body(zs_hbm, src_hbm, dst_hbm, out_hbm,
              srcv, dstv, rows, gsems, ssem, zbuf, acc_sh):
    c = lax.axis_index("c")
    s = lax.axis_index("s")

    pltpu.sync_copy(src_hbm.at[c * NS + s], srcv)
    pltpu.sync_copy(dst_hbm.at[s], dstv)

    def wait_gather(j, b):
        pltpu.make_async_copy(zs_hbm.at[srcv.at[j]], rows.at[b],
                              gsems.at[b]).wait()

    def fire_gather(j, b):
        pltpu.async_copy(zs_hbm.at[srcv.at[j]], rows.at[b], gsems.at[b])

    def fire_scatter(j, b):
        pltpu.async_copy(rows.at[b], acc_sh.at[dstv.at[j]], ssem, add=True)

    def wait_scatter(j, b):
        pltpu.make_async_copy(rows.at[b], acc_sh.at[dstv.at[j]], ssem).wait()

    fire_gather(0, 0)
    fire_gather(1, 1)
    _zero_rows(zbuf, ZR, FW)
    base = s * STRIPE
    for t in range(STRIPE // ZR):
        pltpu.sync_copy(zbuf, acc_sh.at[pl.ds(base + t * ZR, ZR)])
    plsc.subcore_barrier()
    for u in range(2):
        wait_gather(u, u)
        fire_scatter(u, u)
        fire_gather(u + 2, u + 2)

    def block(blk, carry):
        j0 = 2 + blk * NBUF
        for u in range(NBUF):
            j = j0 + u
            b = (2 + u) % NBUF
            bp = u % NBUF
            wait_scatter(j - 2, bp)
            fire_gather(j + 2, bp)
            wait_gather(j, b)
            fire_scatter(j, b)
        return carry
    lax.fori_loop(0, (NCHS - 4) // NBUF, block, 0)

    for u in range(2):
        j = NCHS - 2 + u
        b = j % NBUF
        wait_gather(j, b)
        fire_scatter(j, b)
        wait_scatter(j - 2, (j - 2) % NBUF)
    for u in range(2):
        j = NCHS - 2 + u
        wait_scatter(j, j % NBUF)

    plsc.subcore_barrier()
    pltpu.sync_copy(acc_sh.at[pl.ds(base, STRIPE)],
                    out_hbm.at[pl.ds(base, STRIPE), pl.ds(c * FW, FW)])


def _sc_mesh():
    return plsc.VectorSubcoreMesh(core_axis_name="c", subcore_axis_name="s",
                                  num_cores=NC, num_subcores=NS)


def _deg_call(dst3):
    fn = pl.kernel(
        _deg_body,
        out_type=jax.ShapeDtypeStruct((NPAD, H), jnp.float32),
        mesh=_sc_mesh(),
        scratch_types=[
            pltpu.VMEM((NCHD, KD), jnp.int32),
            pltpu.VMEM((KD, DW), jnp.float32),
            pltpu.VMEM((ZR, DW), jnp.float32),
            pltpu.VMEM_SHARED((NPAD, DW), jnp.float32),
            pltpu.SemaphoreType.DMA,
        ],
        compiler_params=pltpu.CompilerParams(use_tc_tiling_on_sc=False),
        name="sc_degree",
    )
    return fn(dst3)


def _seg_call(zs2n, src4, dst3):
    fn = pl.kernel(
        _seg_body,
        out_type=jax.ShapeDtypeStruct((NPAD, H), jnp.float32),
        mesh=_sc_mesh(),
        scratch_types=[
            pltpu.VMEM((NCHS, KS), jnp.int32),
            pltpu.VMEM((NCHS, KS), jnp.int32),
            pltpu.VMEM((NBUF, KS, FW), jnp.float32),
            pltpu.SemaphoreType.DMA((NBUF,)),
            pltpu.SemaphoreType.DMA,
            pltpu.VMEM((ZR, FW), jnp.float32),
            pltpu.VMEM_SHARED((NPAD, FW), jnp.float32),
        ],
        compiler_params=pltpu.CompilerParams(use_tc_tiling_on_sc=False),
        name="sc_segsum",
    )
    return fn(zs2n, src4, dst3)


B = 5000
GRID = N // B
_MM = (((1,), (1,)), ((), ()))


def _row_spec(cols):
    return pl.BlockSpec((B, cols), lambda i: (i, 0))


def _full_spec(r, cols):
    return pl.BlockSpec((r, cols), lambda i: (0, 0))


def _a_body(degp, x, w, dis_o, zs_o):
    d = degp[...]
    deg = d[:, 0:1] + d[:, FW:FW + 1] + 1.0
    dis = lax.rsqrt(deg)
    z = lax.dot_general(x[...], w[...], _MM, preferred_element_type=jnp.float32)
    dis_o[...] = jnp.broadcast_to(dis, (B, H))
    zs_o[...] = dis * z


def _b_body(u, zs, dis, b, w, h_o, zs1_o):
    h = jnp.maximum(dis[...] * (u[...] + zs[...]) + b[...], 0.0)
    h_o[...] = h
    z1 = lax.dot_general(h, w[...], _MM, preferred_element_type=jnp.float32)
    zs1_o[...] = dis[...] * z1


def _c_body(u, zs, dis, b, h0, w, zs2_o):
    h1 = jnp.maximum(dis[...] * (u[...] + zs[...]) + b[...], 0.0) + h0[...]
    z2 = lax.dot_general(h1, w[...], _MM, preferred_element_type=jnp.float32)
    zs2_o[...] = dis[...] * z2


def _d_body(u, zs, dis, b, wm, bm, out_o):
    h2 = dis[...] * (u[...] + zs[...]) + b[...]
    out_o[...] = (
        lax.dot_general(h2, wm[...], _MM, preferred_element_type=jnp.float32)
        + bm[...]
    )


def _stage_a(degp, x, w0):
    return pl.pallas_call(
        _a_body,
        grid=(GRID,),
        in_specs=[_row_spec(H), _row_spec(D), _full_spec(H, D)],
        out_specs=[_row_spec(H), _row_spec(H)],
        out_shape=[jax.ShapeDtypeStruct((N, H), jnp.float32),
                   jax.ShapeDtypeStruct((N, H), jnp.float32)],
    )(degp, x, w0)


def _stage_b(u, zs, dis, b0, w1):
    return pl.pallas_call(
        _b_body,
        grid=(GRID,),
        in_specs=[_row_spec(H), _row_spec(H), _row_spec(H),
                  _full_spec(1, H), _full_spec(H, H)],
        out_specs=[_row_spec(H), _row_spec(H)],
        out_shape=[jax.ShapeDtypeStruct((N, H), jnp.float32),
                   jax.ShapeDtypeStruct((N, H), jnp.float32)],
    )(u, zs, dis, b0, w1)


def _stage_c(u, zs, dis, b1, h0, w2):
    return pl.pallas_call(
        _c_body,
        grid=(GRID,),
        in_specs=[_row_spec(H), _row_spec(H), _row_spec(H),
                  _full_spec(1, H), _row_spec(H), _full_spec(H, H)],
        out_specs=_row_spec(H),
        out_shape=jax.ShapeDtypeStruct((N, H), jnp.float32),
    )(u, zs, dis, b1, h0, w2)


def _stage_d(u, zs, dis, b2, wm, bm):
    return pl.pallas_call(
        _d_body,
        grid=(GRID,),
        in_specs=[_row_spec(H), _row_spec(H), _row_spec(H),
                  _full_spec(1, H), _full_spec(C, H), _full_spec(1, C)],
        out_specs=pl.BlockSpec((B, C), lambda i: (i, 0)),
        out_shape=jax.ShapeDtypeStruct((N, C), jnp.float32),
    )(u, zs, dis, b2, wm, bm)


def kernel(X, A, W0, b0, W1, b1, W2, b2, Wm, bm):
    src, dst = A[0], A[1]
    src_t = (2 * src).reshape(NS, NCHS, KS)
    src4 = jnp.concatenate([src_t, src_t + 1], axis=0)
    dst3s = dst.reshape(NS, NCHS, KS)
    dst3d = dst.reshape(NW, NCHD, KD)

    degp = _deg_call(dst3d)
    dis_b, zs0 = _stage_a(degp, X, W0)

    u0 = _seg_call(zs0.reshape(2 * N, FW), src4, dst3s)
    h0, zs1 = _stage_b(u0, zs0, dis_b, b0.reshape(1, H), W1)

    u1 = _seg_call(zs1.reshape(2 * N, FW), src4, dst3s)
    zs2 = _stage_c(u1, zs1, dis_b, b1.reshape(1, H), h0, W2)

    u2 = _seg_call(zs2.reshape(2 * N, FW), src4, dst3s)
    return _stage_d(u2, zs2, dis_b, b2.reshape(1, H), Wm, bm.reshape(1, C))


# speedup vs baseline: 1.7588x; 1.0043x over previous
import jax
import jax.numpy as jnp
from jax import lax
from jax.experimental import pallas as pl
from jax.experimental.pallas import tpu as pltpu
from jax.experimental.pallas import tpu_sc as plsc

N = 10000
E = 320000
D = 128
H = 128
C = 40

NC = 2
NS = 16
NW = NC * NS
FW = H // NC
NPAD = 10112
STRIPE = NPAD // NS
ZR = 79
DW = 16

EPT = E // NS
KS = 125
NCHS = EPT // KS
NBUF = 4

EPW = E // NW
KD = 125
NCHD = EPW // KD


def _zero_rows(ref, rows, cols):
    def body(i, carry):
        for j in range(cols // 16):
            ref[i, pl.ds(j * 16, 16)] = jnp.zeros((16,), jnp.float32)
        return carry
    lax.fori_loop(0, rows, body, 0)


def _deg_body(dst_hbm, out_hbm, dstv, onesv, zbuf, acc_sh, ssem):
    c = lax.axis_index("c")
    s = lax.axis_index("s")
    wid = c * NS + s

    pltpu.sync_copy(dst_hbm.at[wid], dstv)

    def fill_ones(i, carry):
        onesv[i, pl.ds(0, 16)] = jnp.ones((16,), jnp.float32)
        return carry
    lax.fori_loop(0, KD, fill_ones, 0)
    _zero_rows(zbuf, ZR, DW)
    base = s * STRIPE
    for t in range(STRIPE // ZR):
        pltpu.sync_copy(zbuf, acc_sh.at[pl.ds(base + t * ZR, ZR)])
    plsc.subcore_barrier()

    def chunk(j, carry):
        pltpu.async_copy(onesv, acc_sh.at[dstv.at[j]], ssem, add=True)
        return carry
    lax.fori_loop(0, NCHD, chunk, 0)

    def drain(j, carry):
        pltpu.make_async_copy(onesv, acc_sh.at[dstv.at[j]], ssem).wait()
        return carry
    lax.fori_loop(0, NCHD, drain, 0)
    plsc.subcore_barrier()
    pltpu.sync_copy(acc_sh.at[pl.ds(base, STRIPE)],
                    out_hbm.at[pl.ds(base, STRIPE), pl.ds(c * FW, DW)])


def _seg_body(zs_hbm, src_hbm, dst_hbm, out_hbm,
              srcv, dstv, rows, gsems, ssem, zbuf, acc_sh):
    c = lax.axis_index("c")
    s = lax.axis_index("s")

    pltpu.sync_copy(src_hbm.at[c * NS + s], srcv)
    pltpu.sync_copy(dst_hbm.at[s], dstv)

    def wait_gather(j, b):
        pltpu.make_async_copy(zs_hbm.at[srcv.at[j]], rows.at[b],
                              gsems.at[b]).wait()

    def fire_gather(j, b):
        pltpu.async_copy(zs_hbm.at[srcv.at[j]], rows.at[b], gsems.at[b])

    def fire_scatter(j, b):
        pltpu.async_copy(rows.at[b], acc_sh.at[dstv.at[j]], ssem, add=True)

    def wait_scatter(j, b):
        pltpu.make_async_copy(rows.at[b], acc_sh.at[dstv.at[j]], ssem).wait()

    fire_gather(0, 0)
    fire_gather(1, 1)
    fire_gather(2, 2)
    _zero_rows(zbuf, ZR, FW)
    base = s * STRIPE
    for t in range(STRIPE // ZR):
        pltpu.sync_copy(zbuf, acc_sh.at[pl.ds(base + t * ZR, ZR)])
    plsc.subcore_barrier()
    fire_gather(3, 3)
    wait_gather(0, 0)
    fire_scatter(0, 0)

    def block(blk, carry):
        j0 = 1 + blk * NBUF
        for u in range(NBUF):
            j = j0 + u
            b = (1 + u) % NBUF
            wait_scatter(j - 1, u % NBUF)
            fire_gather(j + 3, u % NBUF)
            wait_gather(j, b)
            fire_scatter(j, b)
        return carry
    lax.fori_loop(0, (NCHS - 4) // NBUF, block, 0)

    for u in range(3):
        j = NCHS - 3 + u
        b = j % NBUF
        wait_scatter(j - 1, (j - 1) % NBUF)
        wait_gather(j, b)
        fire_scatter(j, b)
    wait_scatter(NCHS - 1, (NCHS - 1) % NBUF)

    plsc.subcore_barrier()
    pltpu.sync_copy(acc_sh.at[pl.ds(base, STRIPE)],
                    out_hbm.at[pl.ds(base, STRIPE), pl.ds(c * FW, FW)])


def _sc_mesh():
    return plsc.VectorSubcoreMesh(core_axis_name="c", subcore_axis_name="s",
                                  num_cores=NC, num_subcores=NS)


def _deg_call(dst3):
    fn = pl.kernel(
        _deg_body,
        out_type=jax.ShapeDtypeStruct((NPAD, H), jnp.float32),
        mesh=_sc_mesh(),
        scratch_types=[
            pltpu.VMEM((NCHD, KD), jnp.int32),
            pltpu.VMEM((KD, DW), jnp.float32),
            pltpu.VMEM((ZR, DW), jnp.float32),
            pltpu.VMEM_SHARED((NPAD, DW), jnp.float32),
            pltpu.SemaphoreType.DMA,
        ],
        compiler_params=pltpu.CompilerParams(use_tc_tiling_on_sc=False),
        name="sc_degree",
    )
    return fn(dst3)


def _seg_call(zs2n, src4, dst3):
    fn = pl.kernel(
        _seg_body,
        out_type=jax.ShapeDtypeStruct((NPAD, H), jnp.float32),
        mesh=_sc_mesh(),
        scratch_types=[
            pltpu.VMEM((NCHS, KS), jnp.int32),
            pltpu.VMEM((NCHS, KS), jnp.int32),
            pltpu.VMEM((NBUF, KS, FW), jnp.float32),
            pltpu.SemaphoreType.DMA((NBUF,)),
            pltpu.SemaphoreType.DMA,
            pltpu.VMEM((ZR, FW), jnp.float32),
            pltpu.VMEM_SHARED((NPAD, FW), jnp.float32),
        ],
        compiler_params=pltpu.CompilerParams(use_tc_tiling_on_sc=False),
        name="sc_segsum",
    )
    return fn(zs2n, src4, dst3)


B = 5000
GRID = N // B
_MM = (((1,), (1,)), ((), ()))


def _row_spec(cols):
    return pl.BlockSpec((B, cols), lambda i: (i, 0))


def _full_spec(r, cols):
    return pl.BlockSpec((r, cols), lambda i: (0, 0))


def _a_body(degp, x, w, dis_o, zs_o):
    d = degp[...]
    deg = d[:, 0:1] + d[:, FW:FW + 1] + 1.0
    dis = lax.rsqrt(deg)
    z = lax.dot_general(x[...], w[...], _MM, preferred_element_type=jnp.float32)
    dis_o[...] = jnp.broadcast_to(dis, (B, H))
    zs_o[...] = dis * z


def _b_body(u, zs, dis, b, w, h_o, zs1_o):
    h = jnp.maximum(dis[...] * (u[...] + zs[...]) + b[...], 0.0)
    h_o[...] = h
    z1 = lax.dot_general(h, w[...], _MM, preferred_element_type=jnp.float32)
    zs1_o[...] = dis[...] * z1


def _c_body(u, zs, dis, b, h0, w, zs2_o):
    h1 = jnp.maximum(dis[...] * (u[...] + zs[...]) + b[...], 0.0) + h0[...]
    z2 = lax.dot_general(h1, w[...], _MM, preferred_element_type=jnp.float32)
    zs2_o[...] = dis[...] * z2


def _d_body(u, zs, dis, b, wm, bm, out_o):
    h2 = dis[...] * (u[...] + zs[...]) + b[...]
    out_o[...] = (
        lax.dot_general(h2, wm[...], _MM, preferred_element_type=jnp.float32)
        + bm[...]
    )


def _stage_a(degp, x, w0):
    return pl.pallas_call(
        _a_body,
        grid=(GRID,),
        in_specs=[_row_spec(H), _row_spec(D), _full_spec(H, D)],
        out_specs=[_row_spec(H), _row_spec(H)],
        out_shape=[jax.ShapeDtypeStruct((N, H), jnp.float32),
                   jax.ShapeDtypeStruct((N, H), jnp.float32)],
    )(degp, x, w0)


def _stage_b(u, zs, dis, b0, w1):
    return pl.pallas_call(
        _b_body,
        grid=(GRID,),
        in_specs=[_row_spec(H), _row_spec(H), _row_spec(H),
                  _full_spec(1, H), _full_spec(H, H)],
        out_specs=[_row_spec(H), _row_spec(H)],
        out_shape=[jax.ShapeDtypeStruct((N, H), jnp.float32),
                   jax.ShapeDtypeStruct((N, H), jnp.float32)],
    )(u, zs, dis, b0, w1)


def _stage_c(u, zs, dis, b1, h0, w2):
    return pl.pallas_call(
        _c_body,
        grid=(GRID,),
        in_specs=[_row_spec(H), _row_spec(H), _row_spec(H),
                  _full_spec(1, H), _row_spec(H), _full_spec(H, H)],
        out_specs=_row_spec(H),
        out_shape=jax.ShapeDtypeStruct((N, H), jnp.float32),
    )(u, zs, dis, b1, h0, w2)


def _stage_d(u, zs, dis, b2, wm, bm):
    return pl.pallas_call(
        _d_body,
        grid=(GRID,),
        in_specs=[_row_spec(H), _row_spec(H), _row_spec(H),
                  _full_spec(1, H), _full_spec(C, H), _full_spec(1, C)],
        out_specs=pl.BlockSpec((B, C), lambda i: (i, 0)),
        out_shape=jax.ShapeDtypeStruct((N, C), jnp.float32),
    )(u, zs, dis, b2, wm, bm)


def kernel(X, A, W0, b0, W1, b1, W2, b2, Wm, bm):
    src, dst = A[0], A[1]
    src_t = (2 * src).reshape(NS, NCHS, KS)
    src4 = jnp.concatenate([src_t, src_t + 1], axis=0)
    dst3s = dst.reshape(NS, NCHS, KS)
    dst3d = dst.reshape(NW, NCHD, KD)

    degp = _deg_call(dst3d)
    dis_b, zs0 = _stage_a(degp, X, W0)

    u0 = _seg_call(zs0.reshape(2 * N, FW), src4, dst3s)
    h0, zs1 = _stage_b(u0, zs0, dis_b, b0.reshape(1, H), W1)

    u1 = _seg_call(zs1.reshape(2 * N, FW), src4, dst3s)
    zs2 = _stage_c(u1, zs1, dis_b, b1.reshape(1, H), h0, W2)

    u2 = _seg_call(zs2.reshape(2 * N, FW), src4, dst3s)
    return _stage_d(u2, zs2, dis_b, b2.reshape(1, H), Wm, bm.reshape(1, C))
